# baseline (speedup 1.0000x reference)
import math

import jax
import jax.numpy as jnp
from jax.experimental import pallas as pl
from jax.experimental.pallas import tpu as pltpu
from jax.experimental.pallas import tpu_sc as plsc

FACIL_RATE = 2.0
FIRE_THRESH = 0.85
MAX_GATE = 8.0

_INV_SQRT2 = 1.0 / math.sqrt(2.0)

_ROW_BLOCK = 4096
_SCALE_BLOCK = 2048
_D = 1024
_N_ROWS = 4 * 8192

_N_BUF = 512
_N_SC_WORKERS = 32
_ROWS_PER_W = _N_BUF // _N_SC_WORKERS
_L = 16


def _sum_kernel(x_ref, out_ref, acc_ref):
    i = pl.program_id(0)

    def body(j, carry):
        s1, s2 = carry
        xx = x_ref[pl.ds(j * 8, 8), :]
        e = jax.lax.erf(xx * _INV_SQRT2)
        return (s1 + xx, s2 + xx * e)

    zero = jnp.zeros((8, _D), jnp.float32)
    s1, s2 = jax.lax.fori_loop(
        0, _ROW_BLOCK // 8, body, (zero, zero), unroll=4)
    part = 0.5 * (s1 + s2)

    @pl.when(i == 0)
    def _init():
        acc_ref[...] = part

    @pl.when(i > 0)
    def _acc():
        acc_ref[...] += part

    @pl.when(i == pl.num_programs(0) - 1)
    def _emit():
        total = acc_ref[...]
        m = jnp.sum(total, axis=0, keepdims=True) * (1.0 / float(_N_ROWS))
        norm = jnp.sqrt(jnp.sum(m * m))
        out_ref[...] = m / jnp.maximum(norm, 1e-12)


def _compute_q(x2):
    n_rows = x2.shape[0]
    grid = n_rows // _ROW_BLOCK
    return pl.pallas_call(
        _sum_kernel,
        grid=(grid,),
        in_specs=[pl.BlockSpec((_ROW_BLOCK, _D), lambda i: (i, 0))],
        out_specs=pl.BlockSpec((1, _D), lambda i: (0, 0)),
        out_shape=jax.ShapeDtypeStruct((1, _D), jnp.float32),
        scratch_shapes=[pltpu.VMEM((8, _D), jnp.float32)],
        compiler_params=pltpu.CompilerParams(
            dimension_semantics=("arbitrary",)),
    )(x2)


def _full(v, dtype=jnp.float32):
    return jnp.full((_L,), v, dtype=dtype)


def _vsum(vec):
    s = vec[0]
    for i in range(1, _L):
        s = s + vec[i]
    return s


def _vmax(vec):
    s = vec[0]
    for i in range(1, _L):
        s = jnp.maximum(s, vec[i])
    return s


def _sc_stats_body(q_hbm, buf_hbm, mask_hbm, out_hbm,
                   q_v, buf_v, maskf_v, stage_v, sem):
    wid = jax.lax.axis_index("s") * 2 + jax.lax.axis_index("c")
    base_row = wid * _ROWS_PER_W

    c1 = pltpu.async_copy(q_hbm, q_v, sem)
    c2 = pltpu.async_copy(
        buf_hbm.at[pl.ds(base_row * 1, _ROWS_PER_W), :], buf_v, sem)
    c3 = pltpu.async_copy(
        mask_hbm.at[pl.ds(base_row * 1, _ROWS_PER_W)], maskf_v, sem)
    c1.wait()
    c2.wait()
    c3.wait()

    lanes = jax.lax.broadcasted_iota(jnp.int32, (_L,), 0)

    def dot_body(c, accs):
        b = c * _L
        qc = q_v[0, pl.ds(b, _L)]
        return tuple(accs[j] + buf_v[j, pl.ds(b, _L)] * qc
                     for j in range(_ROWS_PER_W))

    zeros = tuple(jnp.zeros((_L,), jnp.float32) for _ in range(_ROWS_PER_W))
    accs = jax.lax.fori_loop(0, _D // _L, dot_body, zeros, unroll=2)
    simvec = jnp.zeros((_L,), jnp.float32)
    for j in range(_ROWS_PER_W):
        simvec = jnp.where(lanes == j, _full(0.0) + _vsum(accs[j]), simvec)

    mk = maskf_v[...] > 0.5
    sm = jnp.where(mk, simvec, _full(-1.0))
    cmax = _vmax(sm)
    lanei = jnp.int32(_L - 1)
    for i2 in range(_L - 1, -1, -1):
        lanei = jnp.where(sm[i2] == cmax, jnp.int32(i2), lanei)
    cidx = base_row + lanei
    csum = _vsum(jnp.where(mk, simvec, _full(0.0)))
    ccnt = _vsum(maskf_v[...])

    statv = jnp.where(lanes == 0, _full(0.0) + cmax,
            jnp.where(lanes == 1, _full(0.0) + cidx.astype(jnp.float32),
            jnp.where(lanes == 2, _full(0.0) + csum,
            jnp.where(lanes == 3, _full(0.0) + ccnt, _full(0.0)))))
    stage_v[...] = statv
    pltpu.sync_copy(stage_v, out_hbm.at[wid])


def _compute_stats_sc(q, buf, mask_f):
    mesh = plsc.VectorSubcoreMesh(core_axis_name="c", subcore_axis_name="s")
    fn = pl.kernel(
        _sc_stats_body,
        out_type=jax.ShapeDtypeStruct((_N_SC_WORKERS, _L), jnp.float32),
        mesh=mesh,
        scratch_types=[
            pltpu.VMEM((1, _D), jnp.float32),
            pltpu.VMEM((_ROWS_PER_W, _D), jnp.float32),
            pltpu.VMEM((_ROWS_PER_W,), jnp.float32),
            pltpu.VMEM((_L,), jnp.float32),
            pltpu.SemaphoreType.DMA,
        ],
    )
    return fn(q, buf, mask_f)


def _scale_kernel(scal_ref, stats_ref, facil_ref, x_ref, out_ref, gate_ref):
    i = pl.program_id(0)

    @pl.when(i == 0)
    def _merge_gate():
        stats = stats_ref[...]
        wmax = stats[:, 0:1]
        widx = stats[:, 1:2]
        gmax = jnp.max(wmax)
        sel = wmax == gmax
        gidx_f = jnp.min(jnp.where(sel, widx, jnp.float32(2.0 ** 30)))
        gidx = gidx_f.astype(jnp.int32)
        gsum = jnp.sum(stats[:, 2:3])
        gcnt = jnp.sum(stats[:, 3:4])

        k_gate = jnp.clip(jnp.exp(scal_ref[0, 0]), 0.01, 5.0)
        sharpness = jnp.clip(jnp.exp(scal_ref[0, 1]), 0.5, 20.0)
        mean_others = (gsum - gmax) / jnp.maximum(gcnt - 1.0, 1.0)
        contrast = jnp.where(gcnt > 1.0, gmax - mean_others, 0.0)
        fire_mult = jnp.where(gmax > FIRE_THRESH, FACIL_RATE, 1.0)
        fiota = jax.lax.broadcasted_iota(jnp.int32, (1, _N_BUF), 1)
        facil_level = jnp.sum(
            jnp.where(fiota == gidx, facil_ref[...], 0.0)) * fire_mult
        selectivity = jax.nn.sigmoid(sharpness * contrast)
        gate = jnp.minimum(1.0 + k_gate * (facil_level - 1.0) * selectivity,
                           MAX_GATE)
        gate_ref[0, 0] = 0.5 * gate

    hg = gate_ref[0, 0]
    x = x_ref[...]
    a = hg * x
    out_ref[...] = a + a * jax.lax.erf(x * _INV_SQRT2)


def _scale(x2, scal, stats, facil2):
    n_rows = x2.shape[0]
    grid = n_rows // _SCALE_BLOCK
    return pl.pallas_call(
        _scale_kernel,
        grid=(grid,),
        in_specs=[
            pl.BlockSpec(memory_space=pltpu.SMEM),
            pl.BlockSpec((_N_SC_WORKERS, _L), lambda i: (0, 0)),
            pl.BlockSpec((1, _N_BUF), lambda i: (0, 0)),
            pl.BlockSpec((_SCALE_BLOCK, _D), lambda i: (i, 0)),
        ],
        out_specs=pl.BlockSpec((_SCALE_BLOCK, _D), lambda i: (i, 0)),
        out_shape=jax.ShapeDtypeStruct((n_rows, _D), jnp.float32),
        scratch_shapes=[pltpu.SMEM((1, 1), jnp.float32)],
        compiler_params=pltpu.CompilerParams(
            dimension_semantics=("arbitrary",)),
    )(scal, stats, facil2, x2)


def kernel(x, log_k_gate, log_sharpness, buf, facil, mask):
    orig_shape = x.shape
    x2 = x.reshape(-1, x.shape[-1])

    q = _compute_q(x2)
    mask_f = mask.astype(jnp.float32)
    stats = _compute_stats_sc(q, buf, mask_f)

    scal = jnp.stack([log_k_gate, log_sharpness]).reshape(1, 2)
    facil2 = facil.reshape(1, -1)
    out = _scale(x2, scal, stats, facil2)
    return out.reshape(orig_shape)
